# baseline (speedup 1.0000x reference)
import functools

import jax
import jax.numpy as jnp
from jax import lax
from jax.experimental import pallas as pl
from jax.experimental.pallas import tpu as pltpu
from jax.experimental.pallas import tpu_sc as plsc

MODEL_DIM = 128
MAX_LENGTH = 128
BATCH = 4096
SEQ_LEN = 128

_INFO = plsc.get_sparse_core_info()
NC = _INFO.num_cores
NS = _INFO.num_subcores
NW = NC * NS
TOTAL_ROWS = BATCH * SEQ_LEN
ROWS_PER_W = TOTAL_ROWS // NW
CHUNK = 128
SROWS = 64
LROWS = CHUNK - SROWS
NCHUNKS = ROWS_PER_W // CHUNK
NBUF = 4
AHEAD = 2
LANES = 16


def _make_kernel():
    mesh = plsc.VectorSubcoreMesh(core_axis_name="c", subcore_axis_name="s")

    @functools.partial(
        pl.kernel,
        mesh=mesh,
        out_type=jax.ShapeDtypeStruct((TOTAL_ROWS, MODEL_DIM), jnp.float32),
        scratch_types=[
            pltpu.VMEM((NCHUNKS, SROWS), jnp.int32),
            pltpu.VMEM((NCHUNKS, LROWS), jnp.int32),
            pltpu.VMEM((NBUF * CHUNK, MODEL_DIM), jnp.float32),
            pltpu.VMEM((MAX_LENGTH, MODEL_DIM), jnp.float32),
            pltpu.VMEM_SHARED((MAX_LENGTH, MODEL_DIM), jnp.float32),
            pltpu.SemaphoreType.DMA,
            pltpu.SemaphoreType.DMA,
        ],
        compiler_params=pltpu.CompilerParams(needs_layout_passes=False),
    )
    def gather_kernel(idx_s_hbm, idx_l_hbm, table_hbm, out_hbm,
                      idx_s, idx_l, rows_v, table_v, table_sh,
                      g_sem, w_sem):
        c = lax.axis_index("c")
        s = lax.axis_index("s")
        wid = s * NC + c
        base = wid * ROWS_PER_W

        @pl.when(s == 0)
        def _():
            pltpu.sync_copy(table_hbm, table_sh)

        pltpu.sync_copy(table_hbm, table_v)
        pltpu.sync_copy(idx_s_hbm.at[wid], idx_s)
        pltpu.sync_copy(idx_l_hbm.at[wid], idx_l)
        plsc.subcore_barrier()

        for b in range(AHEAD):
            pltpu.async_copy(
                table_sh.at[idx_s.at[b]],
                rows_v.at[pl.ds(b * CHUNK, SROWS)],
                g_sem,
            )

        iota16 = lax.iota(jnp.int32, LANES)

        def chunk_step(i, carry):
            buf = lax.rem(i, NBUF)
            rbase = buf * CHUNK

            for j in range(LROWS // LANES):
                idxv = idx_l[i, pl.ds(j * LANES, LANES)]
                for b in range(LANES):
                    idx_r = lax.gather(
                        idxv,
                        jnp.full((LANES, 1), b, jnp.int32),
                        lax.GatherDimensionNumbers(
                            offset_dims=(),
                            collapsed_slice_dims=(0,),
                            start_index_map=(0,),
                        ),
                        (1,),
                        mode=lax.GatherScatterMode.PROMISE_IN_BOUNDS,
                    )
                    row = rbase + SROWS + j * LANES + b
                    for g in range(MODEL_DIM // LANES):
                        colv = iota16 + g * LANES
                        vals = plsc.load_gather(table_v, [idx_r, colv])
                        rows_v[row, pl.ds(g * LANES, LANES)] = vals

            pltpu.make_async_copy(
                table_sh.at[idx_s.at[0]],
                rows_v.at[pl.ds(0, SROWS)],
                g_sem,
            ).wait()

            pltpu.async_copy(
                rows_v.at[pl.ds(rbase, CHUNK)],
                out_hbm.at[pl.ds(base + i * CHUNK, CHUNK)],
                w_sem,
            )

            @pl.when(i + AHEAD < NCHUNKS)
            def _():
                nxt = lax.rem(i + AHEAD, NBUF)

                @pl.when(i >= NBUF - AHEAD)
                def _():
                    pltpu.make_async_copy(
                        rows_v.at[pl.ds(0, CHUNK)],
                        out_hbm.at[pl.ds(base, CHUNK)],
                        w_sem,
                    ).wait()

                pltpu.async_copy(
                    table_sh.at[idx_s.at[i + AHEAD]],
                    rows_v.at[pl.ds(nxt * CHUNK, SROWS)],
                    g_sem,
                )

            return carry

        lax.fori_loop(0, NCHUNKS, chunk_step, 0)
        for b in range(NBUF):
            pltpu.make_async_copy(
                rows_v.at[pl.ds(0, CHUNK)],
                out_hbm.at[pl.ds(base, CHUNK)],
                w_sem,
            ).wait()

    return gather_kernel


_kernel_fn = _make_kernel()


@jax.jit
def kernel(span_indices, table):
    idx = span_indices.reshape(NW, NCHUNKS, CHUNK).astype(jnp.int32)
    out = _kernel_fn(idx[:, :, :SROWS], idx[:, :, SROWS:], table)
    return out.reshape(BATCH, SEQ_LEN, MODEL_DIM)
